# baseline (speedup 1.0000x reference)
import functools

import jax
import jax.numpy as jnp
from jax import lax
from jax.experimental import pallas as pl
from jax.experimental.pallas import tpu as pltpu
from jax.experimental.pallas import tpu_sc as plsc

N = 10000
E = 160000
NP = 10240
EP = 163840
NPT = NP // 16

_MESH = dict(core_axis_name="c", subcore_axis_name="s")


def _sc_degrees(e16):
    @functools.partial(
        pl.kernel,
        mesh=plsc.VectorSubcoreMesh(**_MESH),
        out_type=jax.ShapeDtypeStruct((2, NP), jnp.float32),
        scratch_types=[
            pltpu.VMEM((80, 128), jnp.int32),
            pltpu.VMEM((128,), jnp.float32),
            pltpu.VMEM((NPT,), jnp.float32),
            pltpu.VMEM_SHARED((NP,), jnp.float32),
        ],
    )
    def k(e_hbm, out_hbm, idxv, onesv, zv, acc):
        cid = lax.axis_index("c")
        sid = lax.axis_index("s")
        pltpu.sync_copy(e_hbm.at[cid, sid], idxv)
        for i in range(8):
            onesv[pl.ds(i * 16, 16)] = jnp.ones((16,), jnp.float32)

        def zrow(r, c):
            zv[pl.ds(r * 16, 16)] = jnp.zeros((16,), jnp.float32)
            return c

        lax.fori_loop(0, NPT // 16, zrow, 0)
        pltpu.sync_copy(zv, acc.at[pl.ds(sid * NPT, NPT)])
        plsc.subcore_barrier()

        def chunk(j, c):
            pltpu.sync_copy(onesv, acc.at[idxv.at[j]], add=True)
            return c

        lax.fori_loop(0, 80, chunk, 0)
        plsc.subcore_barrier()
        pltpu.sync_copy(acc.at[pl.ds(sid * NPT, NPT)],
                        out_hbm.at[cid, pl.ds(sid * NPT, NPT)])

    return k(e16)


RH = 5120
ACC_R = 5632
ZPT = ACC_R // 16
CPT = RH // 16


def _sc_agg(table, edges, nchunks, per_core_edges, offset_by_core):
    @functools.partial(
        pl.kernel,
        mesh=plsc.VectorSubcoreMesh(**_MESH),
        out_type=jax.ShapeDtypeStruct((2, NP, 128), jnp.float32),
        scratch_types=[
            pltpu.VMEM((nchunks, 128), jnp.int32),
            pltpu.VMEM((nchunks, 128), jnp.int32),
            pltpu.VMEM((nchunks, 128), jnp.int32),
            pltpu.VMEM((128, 128), jnp.float32),
            pltpu.VMEM((128, 128), jnp.float32),
            pltpu.VMEM((32, 128), jnp.float32),
            pltpu.VMEM_SHARED((ACC_R, 128), jnp.float32),
            pltpu.SemaphoreType.DMA,
            pltpu.SemaphoreType.DMA,
        ],
    )
    def k(tab_hbm, e_hbm, out_hbm, srcv, dstAv, dstBv, rows0, rows1,
          zbuf, acc, gsem0, gsem1):
        cid = lax.axis_index("c")
        sid = lax.axis_index("s")
        tid = cid * 16 + sid if per_core_edges else sid
        pltpu.sync_copy(e_hbm.at[0, tid], srcv)
        pltpu.sync_copy(e_hbm.at[1, tid], dstAv)
        pltpu.sync_copy(e_hbm.at[2, tid], dstBv)

        if offset_by_core:
            off = cid * NP

            def addoff(j, c):
                for i in range(8):
                    srcv[j, pl.ds(i * 16, 16)] = srcv[j, pl.ds(i * 16, 16)] + off
                return c

            lax.fori_loop(0, nchunks, addoff, 0)

        def zrow(r, c):
            for i in range(8):
                zbuf[r, pl.ds(i * 16, 16)] = jnp.zeros((16,), jnp.float32)
            return c

        lax.fori_loop(0, 32, zrow, 0)

        for p in range(2):
            dstv = dstAv if p == 0 else dstBv
            for b in range(ZPT // 32):
                pltpu.sync_copy(zbuf,
                                acc.at[pl.ds(sid * ZPT + b * 32, 32)])
            plsc.subcore_barrier()

            pltpu.async_copy(tab_hbm.at[srcv.at[0]], rows0, gsem0)

            def chunk2(jj, c):
                j0 = 2 * jj
                pltpu.async_copy(tab_hbm.at[srcv.at[j0 + 1]], rows1, gsem1)
                pltpu.make_async_copy(tab_hbm.at[srcv.at[j0]], rows0,
                                      gsem0).wait()
                pltpu.sync_copy(rows0, acc.at[dstv.at[j0]], add=True)

                @pl.when(jj < nchunks // 2 - 1)
                def _():
                    pltpu.async_copy(tab_hbm.at[srcv.at[j0 + 2]], rows0, gsem0)

                pltpu.make_async_copy(tab_hbm.at[srcv.at[j0 + 1]], rows1,
                                      gsem1).wait()
                pltpu.sync_copy(rows1, acc.at[dstv.at[j0 + 1]], add=True)
                return c

            lax.fori_loop(0, nchunks // 2, chunk2, 0)
            plsc.subcore_barrier()
            pltpu.sync_copy(acc.at[pl.ds(sid * CPT, CPT)],
                            out_hbm.at[cid, pl.ds(p * RH + sid * CPT, CPT)])
            plsc.subcore_barrier()

    return k(table, edges)


def _tc_prep(degT, Xp):
    bm = 2048

    def body(d_ref, x_ref, xn_ref, n_ref):
        d = d_ref[...]
        nrm = jnp.where(d > 0, lax.rsqrt(jnp.maximum(d, 1.0)), 0.0)
        n_ref[...] = nrm
        ns = nrm[:, 0:1]
        x = x_ref[...]
        xn_ref[0] = x[:, :128] * ns
        xn_ref[1] = x[:, 128:] * ns

    return pl.pallas_call(
        body,
        grid=(NP // bm,),
        in_specs=[
            pl.BlockSpec((bm, 2), lambda i: (i, 0)),
            pl.BlockSpec((bm, 256), lambda i: (i, 0)),
        ],
        out_specs=[
            pl.BlockSpec((2, bm, 128), lambda i: (0, i, 0)),
            pl.BlockSpec((bm, 2), lambda i: (i, 0)),
        ],
        out_shape=[
            jax.ShapeDtypeStruct((2, NP, 128), jnp.float32),
            jax.ShapeDtypeStruct((NP, 2), jnp.float32),
        ],
    )(degT, Xp)


def _tc_mid(agg1, norms, W1a, W1b, b1, W2):
    bm = 2048

    def body(a_ref, n_ref, w1a_ref, w1b_ref, b1_ref, w2_ref, t_ref):
        nrm = n_ref[...]
        ns = nrm[:, 0:1]
        nd = nrm[:, 1:2]
        pre = (jnp.dot(a_ref[0] * nd, w1a_ref[...],
                       preferred_element_type=jnp.float32)
               + jnp.dot(a_ref[1] * nd, w1b_ref[...],
                         preferred_element_type=jnp.float32)
               + b1_ref[...])
        h = jnp.maximum(pre, 0.0)
        t_ref[...] = jnp.dot(h * ns, w2_ref[...],
                             preferred_element_type=jnp.float32)

    return pl.pallas_call(
        body,
        grid=(NP // bm,),
        in_specs=[
            pl.BlockSpec((2, bm, 128), lambda i: (0, i, 0)),
            pl.BlockSpec((bm, 2), lambda i: (i, 0)),
            pl.BlockSpec((128, 256), lambda i: (0, 0)),
            pl.BlockSpec((128, 256), lambda i: (0, 0)),
            pl.BlockSpec((1, 256), lambda i: (0, 0)),
            pl.BlockSpec((256, 128), lambda i: (0, 0)),
        ],
        out_specs=pl.BlockSpec((bm, 128), lambda i: (i, 0)),
        out_shape=jax.ShapeDtypeStruct((NP, 128), jnp.float32),
    )(agg1, norms, W1a, W1b, b1, W2)


def _tc_z(agg2, norms, b2):
    bm = 2048

    def body(a_ref, n_ref, b2_ref, z_ref, zb_ref):
        nd = n_ref[...][:, 1:2]
        z = (a_ref[0] + a_ref[1]) * nd + b2_ref[...]
        z_ref[...] = z
        zb_ref[...] = z.astype(jnp.bfloat16)

    return pl.pallas_call(
        body,
        grid=(NP // bm,),
        in_specs=[
            pl.BlockSpec((2, bm, 128), lambda i: (0, i, 0)),
            pl.BlockSpec((bm, 2), lambda i: (i, 0)),
            pl.BlockSpec((1, 128), lambda i: (0, 0)),
        ],
        out_specs=[pl.BlockSpec((bm, 128), lambda i: (i, 0)),
                   pl.BlockSpec((bm, 128), lambda i: (i, 0))],
        out_shape=[jax.ShapeDtypeStruct((NP, 128), jnp.float32),
                   jax.ShapeDtypeStruct((NP, 128), jnp.bfloat16)],
    )(agg2, norms, b2)


def _tc_decoder(z):
    bm, bn = 512, 1024

    def body(a_ref, b_ref, o_ref):
        o_ref[...] = lax.dot_general(
            a_ref[...], b_ref[...], (((1,), (1,)), ((), ())),
            preferred_element_type=jnp.float32)

    return pl.pallas_call(
        body,
        grid=(pl.cdiv(N, bm), pl.cdiv(N, bn)),
        in_specs=[
            pl.BlockSpec((bm, 128), lambda i, j: (i, 0)),
            pl.BlockSpec((bn, 128), lambda i, j: (j, 0)),
        ],
        out_specs=pl.BlockSpec((bm, bn), lambda i, j: (i, j)),
        out_shape=jax.ShapeDtypeStruct((N, N), jnp.float32),
    )(z, z)


def kernel(g, features, W1, b1, W2, b2):
    src = g[0].astype(jnp.int32)
    dst = g[1].astype(jnp.int32)
    pad = EP - E
    srcp = jnp.concatenate([src, jnp.full((pad,), N, jnp.int32)])
    dstp = jnp.concatenate(
        [dst, N + (jnp.arange(pad, dtype=jnp.int32) % (NP - N))])

    e16 = jnp.stack([srcp, dstp]).reshape(2, 16, 80, 128)
    deg = _sc_degrees(e16)
    degT = deg.T

    Xp = jnp.pad(features, ((0, NP - N), (0, 0)))
    xn, norms = _tc_prep(degT, Xp)

    dum = RH + (jnp.arange(EP, dtype=jnp.int32) & 511)
    dstA = jnp.where(dstp < RH, dstp, dum)
    dstB = jnp.where(dstp >= RH, dstp - RH, dum)
    e3 = jnp.stack([srcp, dstA, dstB])

    table1 = xn.reshape(2 * NP, 128)
    agg1 = _sc_agg(table1, e3.reshape(3, 16, 80, 128), 80,
                   per_core_edges=False, offset_by_core=True)

    t = _tc_mid(agg1, norms, W1[:128], W1[128:], b1.reshape(1, 256), W2)

    agg2 = _sc_agg(t, e3.reshape(3, 32, 40, 128), 40,
                   per_core_edges=True, offset_by_core=False)

    _, zb = _tc_z(agg2, norms, b2.reshape(1, 128))
    return _tc_decoder(zb)


# speedup vs baseline: 2.3022x; 2.3022x over previous
import functools

import jax
import jax.numpy as jnp
from jax import lax
from jax.experimental import pallas as pl
from jax.experimental.pallas import tpu as pltpu
from jax.experimental.pallas import tpu_sc as plsc

N = 10000
E = 160000
NP = 10240
EP = 163840
NPT = NP // 16

_MESH = dict(core_axis_name="c", subcore_axis_name="s")


def _sc_degrees(e16):
    @functools.partial(
        pl.kernel,
        mesh=plsc.VectorSubcoreMesh(**_MESH),
        out_type=jax.ShapeDtypeStruct((2, NP), jnp.float32),
        scratch_types=[
            pltpu.VMEM((80, 128), jnp.int32),
            pltpu.VMEM((128,), jnp.float32),
            pltpu.VMEM((NPT,), jnp.float32),
            pltpu.VMEM_SHARED((NP,), jnp.float32),
        ],
    )
    def k(e_hbm, out_hbm, idxv, onesv, zv, acc):
        cid = lax.axis_index("c")
        sid = lax.axis_index("s")
        pltpu.sync_copy(e_hbm.at[cid, sid], idxv)
        for i in range(8):
            onesv[pl.ds(i * 16, 16)] = jnp.ones((16,), jnp.float32)

        def zrow(r, c):
            zv[pl.ds(r * 16, 16)] = jnp.zeros((16,), jnp.float32)
            return c

        lax.fori_loop(0, NPT // 16, zrow, 0)
        pltpu.sync_copy(zv, acc.at[pl.ds(sid * NPT, NPT)])
        plsc.subcore_barrier()

        def chunk(j, c):
            pltpu.sync_copy(onesv, acc.at[idxv.at[j]], add=True)
            return c

        lax.fori_loop(0, 80, chunk, 0)
        plsc.subcore_barrier()
        pltpu.sync_copy(acc.at[pl.ds(sid * NPT, NPT)],
                        out_hbm.at[cid, pl.ds(sid * NPT, NPT)])

    return k(e16)


RH = 5120
ACC_R = 5632
ZPT = ACC_R // 16
CPT = RH // 16


def _sc_agg(table, edges, nchunks, per_core_edges, offset_by_core):
    @functools.partial(
        pl.kernel,
        mesh=plsc.VectorSubcoreMesh(**_MESH),
        out_type=jax.ShapeDtypeStruct((2, NP, 128), jnp.float32),
        scratch_types=[
            pltpu.VMEM((nchunks, 128), jnp.int32),
            pltpu.VMEM((nchunks, 128), jnp.int32),
            pltpu.VMEM((nchunks, 128), jnp.int32),
            pltpu.VMEM((128, 128), jnp.float32),
            pltpu.VMEM((128, 128), jnp.float32),
            pltpu.VMEM((32, 128), jnp.float32),
            pltpu.VMEM_SHARED((ACC_R, 128), jnp.float32),
            pltpu.SemaphoreType.DMA,
            pltpu.SemaphoreType.DMA,
        ],
    )
    def k(tab_hbm, e_hbm, out_hbm, srcv, dstAv, dstBv, rows0, rows1,
          zbuf, acc, gsem0, gsem1):
        cid = lax.axis_index("c")
        sid = lax.axis_index("s")
        tid = cid * 16 + sid if per_core_edges else sid
        pltpu.sync_copy(e_hbm.at[0, tid], srcv)
        pltpu.sync_copy(e_hbm.at[1, tid], dstAv)
        pltpu.sync_copy(e_hbm.at[2, tid], dstBv)

        if offset_by_core:
            off = cid * NP

            def addoff(j, c):
                for i in range(8):
                    srcv[j, pl.ds(i * 16, 16)] = srcv[j, pl.ds(i * 16, 16)] + off
                return c

            lax.fori_loop(0, nchunks, addoff, 0)

        def zrow(r, c):
            for i in range(8):
                zbuf[r, pl.ds(i * 16, 16)] = jnp.zeros((16,), jnp.float32)
            return c

        lax.fori_loop(0, 32, zrow, 0)

        for p in range(2):
            dstv = dstAv if p == 0 else dstBv
            for b in range(ZPT // 32):
                pltpu.sync_copy(zbuf,
                                acc.at[pl.ds(sid * ZPT + b * 32, 32)])
            plsc.subcore_barrier()

            pltpu.async_copy(tab_hbm.at[srcv.at[0]], rows0, gsem0)

            def chunk2(jj, c):
                j0 = 2 * jj
                pltpu.async_copy(tab_hbm.at[srcv.at[j0 + 1]], rows1, gsem1)
                pltpu.make_async_copy(tab_hbm.at[srcv.at[j0]], rows0,
                                      gsem0).wait()
                pltpu.sync_copy(rows0, acc.at[dstv.at[j0]], add=True)

                @pl.when(jj < nchunks // 2 - 1)
                def _():
                    pltpu.async_copy(tab_hbm.at[srcv.at[j0 + 2]], rows0, gsem0)

                pltpu.make_async_copy(tab_hbm.at[srcv.at[j0 + 1]], rows1,
                                      gsem1).wait()
                pltpu.sync_copy(rows1, acc.at[dstv.at[j0 + 1]], add=True)
                return c

            lax.fori_loop(0, nchunks // 2, chunk2, 0)
            plsc.subcore_barrier()
            pltpu.sync_copy(acc.at[pl.ds(sid * CPT, CPT)],
                            out_hbm.at[cid, pl.ds(p * RH + sid * CPT, CPT)])
            plsc.subcore_barrier()

    return k(table, edges)


def _tc_prep(degT, Xp):
    bm = 2048

    def body(d_ref, x_ref, xn_ref, n_ref):
        d = d_ref[...]
        nrm = jnp.where(d > 0, lax.rsqrt(jnp.maximum(d, 1.0)), 0.0)
        n_ref[...] = nrm
        ns = nrm[:, 0:1]
        x = x_ref[...]
        xn_ref[0] = x[:, :128] * ns
        xn_ref[1] = x[:, 128:] * ns

    return pl.pallas_call(
        body,
        grid=(NP // bm,),
        in_specs=[
            pl.BlockSpec((bm, 2), lambda i: (i, 0)),
            pl.BlockSpec((bm, 256), lambda i: (i, 0)),
        ],
        out_specs=[
            pl.BlockSpec((2, bm, 128), lambda i: (0, i, 0)),
            pl.BlockSpec((bm, 2), lambda i: (i, 0)),
        ],
        out_shape=[
            jax.ShapeDtypeStruct((2, NP, 128), jnp.float32),
            jax.ShapeDtypeStruct((NP, 2), jnp.float32),
        ],
    )(degT, Xp)


def _tc_mid(agg1, norms, W1a, W1b, b1, W2):
    bm = 2048

    def body(a_ref, n_ref, w1a_ref, w1b_ref, b1_ref, w2_ref, t_ref):
        nrm = n_ref[...]
        ns = nrm[:, 0:1]
        nd = nrm[:, 1:2]
        pre = (jnp.dot(a_ref[0] * nd, w1a_ref[...],
                       preferred_element_type=jnp.float32)
               + jnp.dot(a_ref[1] * nd, w1b_ref[...],
                         preferred_element_type=jnp.float32)
               + b1_ref[...])
        h = jnp.maximum(pre, 0.0)
        t_ref[...] = jnp.dot(h * ns, w2_ref[...],
                             preferred_element_type=jnp.float32)

    return pl.pallas_call(
        body,
        grid=(NP // bm,),
        in_specs=[
            pl.BlockSpec((2, bm, 128), lambda i: (0, i, 0)),
            pl.BlockSpec((bm, 2), lambda i: (i, 0)),
            pl.BlockSpec((128, 256), lambda i: (0, 0)),
            pl.BlockSpec((128, 256), lambda i: (0, 0)),
            pl.BlockSpec((1, 256), lambda i: (0, 0)),
            pl.BlockSpec((256, 128), lambda i: (0, 0)),
        ],
        out_specs=pl.BlockSpec((bm, 128), lambda i: (i, 0)),
        out_shape=jax.ShapeDtypeStruct((NP, 128), jnp.float32),
    )(agg1, norms, W1a, W1b, b1, W2)


def _tc_z(agg2, norms, b2):
    bm = 2048

    def body(a_ref, n_ref, b2_ref, z_ref):
        nd = n_ref[...][:, 1:2]
        z_ref[...] = (a_ref[0] + a_ref[1]) * nd + b2_ref[...]

    return pl.pallas_call(
        body,
        grid=(NP // bm,),
        in_specs=[
            pl.BlockSpec((2, bm, 128), lambda i: (0, i, 0)),
            pl.BlockSpec((bm, 2), lambda i: (i, 0)),
            pl.BlockSpec((1, 128), lambda i: (0, 0)),
        ],
        out_specs=pl.BlockSpec((bm, 128), lambda i: (i, 0)),
        out_shape=jax.ShapeDtypeStruct((NP, 128), jnp.float32),
    )(agg2, norms, b2)


def _tc_decoder(z):
    bm, bn = 512, 1024

    def body(a_ref, b_ref, o_ref):
        o_ref[...] = lax.dot_general(
            a_ref[...], b_ref[...], (((1,), (1,)), ((), ())),
            preferred_element_type=jnp.float32)

    return pl.pallas_call(
        body,
        grid=(pl.cdiv(N, bm), pl.cdiv(N, bn)),
        in_specs=[
            pl.BlockSpec((bm, 128), lambda i, j: (i, 0)),
            pl.BlockSpec((bn, 128), lambda i, j: (j, 0)),
        ],
        out_specs=pl.BlockSpec((bm, bn), lambda i, j: (i, j)),
        out_shape=jax.ShapeDtypeStruct((N, N), jnp.float32),
    )(z, z)


def kernel(g, features, W1, b1, W2, b2):
    src = g[0].astype(jnp.int32)
    dst = g[1].astype(jnp.int32)
    pad = EP - E
    spread = N + (jnp.arange(pad, dtype=jnp.int32) % (NP - N))
    srcp = jnp.concatenate([src, spread])
    dstp = jnp.concatenate([dst, spread])

    e16 = jnp.stack([srcp, dstp]).reshape(2, 16, 80, 128)
    deg = _sc_degrees(e16)
    degT = deg.T

    Xp = jnp.pad(features, ((0, NP - N), (0, 0)))
    xn, norms = _tc_prep(degT, Xp)

    dum = RH + (jnp.arange(EP, dtype=jnp.int32) & 511)
    dstA = jnp.where(dstp < RH, dstp, dum)
    dstB = jnp.where(dstp >= RH, dstp - RH, dum)
    e3 = jnp.stack([srcp, dstA, dstB])

    table1 = xn.reshape(2 * NP, 128)
    agg1 = _sc_agg(table1, e3.reshape(3, 16, 80, 128), 80,
                   per_core_edges=False, offset_by_core=True)

    t = _tc_mid(agg1, norms, W1[:128], W1[128:], b1.reshape(1, 256), W2)

    agg2 = _sc_agg(t, e3.reshape(3, 32, 40, 128), 40,
                   per_core_edges=True, offset_by_core=False)

    z = _tc_z(agg2, norms, b2.reshape(1, 128))
    return _tc_decoder(z)


# speedup vs baseline: 2.6034x; 1.1309x over previous
import functools

import jax
import jax.numpy as jnp
from jax import lax
from jax.experimental import pallas as pl
from jax.experimental.pallas import tpu as pltpu
from jax.experimental.pallas import tpu_sc as plsc

N = 10000
E = 160000
NP = 10240
EP = 163840
NPT = NP // 16

_MESH = dict(core_axis_name="c", subcore_axis_name="s")


def _sc_degrees(e16):
    @functools.partial(
        pl.kernel,
        mesh=plsc.VectorSubcoreMesh(**_MESH),
        out_type=jax.ShapeDtypeStruct((2, NP), jnp.float32),
        scratch_types=[
            pltpu.VMEM((80, 128), jnp.int32),
            pltpu.VMEM((128,), jnp.float32),
            pltpu.VMEM((NPT,), jnp.float32),
            pltpu.VMEM_SHARED((NP,), jnp.float32),
        ],
    )
    def k(e_hbm, out_hbm, idxv, onesv, zv, acc):
        cid = lax.axis_index("c")
        sid = lax.axis_index("s")
        pltpu.sync_copy(e_hbm.at[cid, sid], idxv)
        for i in range(8):
            onesv[pl.ds(i * 16, 16)] = jnp.ones((16,), jnp.float32)

        def zrow(r, c):
            zv[pl.ds(r * 16, 16)] = jnp.zeros((16,), jnp.float32)
            return c

        lax.fori_loop(0, NPT // 16, zrow, 0)
        pltpu.sync_copy(zv, acc.at[pl.ds(sid * NPT, NPT)])
        plsc.subcore_barrier()

        def chunk(j, c):
            pltpu.sync_copy(onesv, acc.at[idxv.at[j]], add=True)
            return c

        lax.fori_loop(0, 80, chunk, 0)
        plsc.subcore_barrier()
        pltpu.sync_copy(acc.at[pl.ds(sid * NPT, NPT)],
                        out_hbm.at[cid, pl.ds(sid * NPT, NPT)])

    return k(e16)


RH = 5120
ACC_R = 5632
ZPT = ACC_R // 16
CPT = RH // 16


def _sc_agg(table, edges, nchunks, per_core_edges, offset_by_core):
    @functools.partial(
        pl.kernel,
        mesh=plsc.VectorSubcoreMesh(**_MESH),
        out_type=jax.ShapeDtypeStruct((2, NP, 128), jnp.float32),
        scratch_types=[
            pltpu.VMEM((nchunks, 128), jnp.int32),
            pltpu.VMEM((nchunks, 128), jnp.int32),
            pltpu.VMEM((nchunks, 128), jnp.int32),
            pltpu.VMEM((128, 128), jnp.float32),
            pltpu.VMEM((128, 128), jnp.float32),
            pltpu.VMEM((32, 128), jnp.float32),
            pltpu.VMEM_SHARED((ACC_R, 128), jnp.float32),
            pltpu.SemaphoreType.DMA,
            pltpu.SemaphoreType.DMA,
        ],
    )
    def k(tab_hbm, e_hbm, out_hbm, srcv, dstAv, dstBv, rows0, rows1,
          zbuf, acc, gsem0, gsem1):
        cid = lax.axis_index("c")
        sid = lax.axis_index("s")
        tid = cid * 16 + sid if per_core_edges else sid
        pltpu.sync_copy(e_hbm.at[0, tid], srcv)
        pltpu.sync_copy(e_hbm.at[1, tid], dstAv)
        pltpu.sync_copy(e_hbm.at[2, tid], dstBv)

        if offset_by_core:
            off = cid * NP

            def addoff(j, c):
                for i in range(8):
                    srcv[j, pl.ds(i * 16, 16)] = srcv[j, pl.ds(i * 16, 16)] + off
                return c

            lax.fori_loop(0, nchunks, addoff, 0)

        def zrow(r, c):
            for i in range(8):
                zbuf[r, pl.ds(i * 16, 16)] = jnp.zeros((16,), jnp.float32)
            return c

        lax.fori_loop(0, 32, zrow, 0)

        for p in range(2):
            dstv = dstAv if p == 0 else dstBv
            for b in range(ZPT // 32):
                pltpu.sync_copy(zbuf,
                                acc.at[pl.ds(sid * ZPT + b * 32, 32)])
            plsc.subcore_barrier()

            pltpu.async_copy(tab_hbm.at[srcv.at[0]], rows0, gsem0)

            def chunk2(jj, c):
                j0 = 2 * jj
                pltpu.async_copy(tab_hbm.at[srcv.at[j0 + 1]], rows1, gsem1)
                pltpu.make_async_copy(tab_hbm.at[srcv.at[j0]], rows0,
                                      gsem0).wait()
                pltpu.sync_copy(rows0, acc.at[dstv.at[j0]], add=True)

                @pl.when(jj < nchunks // 2 - 1)
                def _():
                    pltpu.async_copy(tab_hbm.at[srcv.at[j0 + 2]], rows0, gsem0)

                pltpu.make_async_copy(tab_hbm.at[srcv.at[j0 + 1]], rows1,
                                      gsem1).wait()
                pltpu.sync_copy(rows1, acc.at[dstv.at[j0 + 1]], add=True)
                return c

            lax.fori_loop(0, nchunks // 2, chunk2, 0)
            plsc.subcore_barrier()
            pltpu.sync_copy(acc.at[pl.ds(sid * CPT, CPT)],
                            out_hbm.at[cid, pl.ds(p * RH + sid * CPT, CPT)])
            plsc.subcore_barrier()

    return k(table, edges)


def _tc_prep(degT, Xp):
    bm = 2048

    def body(d_ref, x_ref, xn_ref, n_ref):
        d = d_ref[...]
        nrm = jnp.where(d > 0, lax.rsqrt(jnp.maximum(d, 1.0)), 0.0)
        n_ref[...] = nrm
        ns = nrm[:, 0:1]
        x = x_ref[...]
        xn_ref[0] = x[:, :128] * ns
        xn_ref[1] = x[:, 128:] * ns

    return pl.pallas_call(
        body,
        grid=(NP // bm,),
        in_specs=[
            pl.BlockSpec((bm, 2), lambda i: (i, 0)),
            pl.BlockSpec((bm, 256), lambda i: (i, 0)),
        ],
        out_specs=[
            pl.BlockSpec((2, bm, 128), lambda i: (0, i, 0)),
            pl.BlockSpec((bm, 2), lambda i: (i, 0)),
        ],
        out_shape=[
            jax.ShapeDtypeStruct((2, NP, 128), jnp.float32),
            jax.ShapeDtypeStruct((NP, 2), jnp.float32),
        ],
    )(degT, Xp)


def _tc_mid(agg1, norms, W1a, W1b, b1, W2):
    bm = 2048

    def body(a_ref, n_ref, w1a_ref, w1b_ref, b1_ref, w2_ref, t_ref):
        nrm = n_ref[...]
        ns = nrm[:, 0:1]
        nd = nrm[:, 1:2]
        pre = (jnp.dot(a_ref[0] * nd, w1a_ref[...],
                       preferred_element_type=jnp.float32)
               + jnp.dot(a_ref[1] * nd, w1b_ref[...],
                         preferred_element_type=jnp.float32)
               + b1_ref[...])
        h = jnp.maximum(pre, 0.0)
        t_ref[...] = jnp.dot(h * ns, w2_ref[...],
                             preferred_element_type=jnp.float32)

    return pl.pallas_call(
        body,
        grid=(NP // bm,),
        in_specs=[
            pl.BlockSpec((2, bm, 128), lambda i: (0, i, 0)),
            pl.BlockSpec((bm, 2), lambda i: (i, 0)),
            pl.BlockSpec((128, 256), lambda i: (0, 0)),
            pl.BlockSpec((128, 256), lambda i: (0, 0)),
            pl.BlockSpec((1, 256), lambda i: (0, 0)),
            pl.BlockSpec((256, 128), lambda i: (0, 0)),
        ],
        out_specs=pl.BlockSpec((bm, 128), lambda i: (i, 0)),
        out_shape=jax.ShapeDtypeStruct((NP, 128), jnp.float32),
    )(agg1, norms, W1a, W1b, b1, W2)


def _tc_z(agg2, norms, b2):
    bm = 2048

    def body(a_ref, n_ref, b2_ref, z_ref):
        nd = n_ref[...][:, 1:2]
        z_ref[...] = (a_ref[0] + a_ref[1]) * nd + b2_ref[...]

    return pl.pallas_call(
        body,
        grid=(NP // bm,),
        in_specs=[
            pl.BlockSpec((2, bm, 128), lambda i: (0, i, 0)),
            pl.BlockSpec((bm, 2), lambda i: (i, 0)),
            pl.BlockSpec((1, 128), lambda i: (0, 0)),
        ],
        out_specs=pl.BlockSpec((bm, 128), lambda i: (i, 0)),
        out_shape=jax.ShapeDtypeStruct((NP, 128), jnp.float32),
    )(agg2, norms, b2)


def _tc_decoder(z):
    bm, bn = 1024, 1024

    def body(a_ref, b_ref, o_ref):
        o_ref[...] = lax.dot_general(
            a_ref[...], b_ref[...], (((1,), (1,)), ((), ())),
            preferred_element_type=jnp.float32)

    return pl.pallas_call(
        body,
        grid=(pl.cdiv(N, bm), pl.cdiv(N, bn)),
        in_specs=[
            pl.BlockSpec((bm, 128), lambda i, j: (i, 0)),
            pl.BlockSpec((bn, 128), lambda i, j: (j, 0)),
        ],
        out_specs=pl.BlockSpec((bm, bn), lambda i, j: (i, j)),
        out_shape=jax.ShapeDtypeStruct((N, N), jnp.float32),
    )(z, z)


def kernel(g, features, W1, b1, W2, b2):
    src = g[0].astype(jnp.int32)
    dst = g[1].astype(jnp.int32)
    pad = EP - E
    spread = N + (jnp.arange(pad, dtype=jnp.int32) % (NP - N))
    srcp = jnp.concatenate([src, spread])
    dstp = jnp.concatenate([dst, spread])

    e16 = jnp.stack([srcp, dstp]).reshape(2, 16, 80, 128)
    deg = _sc_degrees(e16)
    degT = deg.T

    Xp = jnp.pad(features, ((0, NP - N), (0, 0)))
    xn, norms = _tc_prep(degT, Xp)

    dum = RH + (jnp.arange(EP, dtype=jnp.int32) & 511)
    dstA = jnp.where(dstp < RH, dstp, dum)
    dstB = jnp.where(dstp >= RH, dstp - RH, dum)
    e3 = jnp.stack([srcp, dstA, dstB])

    table1 = xn.reshape(2 * NP, 128)
    agg1 = _sc_agg(table1, e3.reshape(3, 16, 80, 128), 80,
                   per_core_edges=False, offset_by_core=True)

    t = _tc_mid(agg1, norms, W1[:128], W1[128:], b1.reshape(1, 256), W2)

    agg2 = _sc_agg(t, e3.reshape(3, 32, 40, 128), 40,
                   per_core_edges=True, offset_by_core=False)

    z = _tc_z(agg2, norms, b2.reshape(1, 128))
    return _tc_decoder(z)


# speedup vs baseline: 2.7139x; 1.0424x over previous
import functools

import jax
import jax.numpy as jnp
from jax import lax
from jax.experimental import pallas as pl
from jax.experimental.pallas import tpu as pltpu
from jax.experimental.pallas import tpu_sc as plsc

N = 10000
E = 160000
NP = 10240
EP = 163840
NPT = NP // 16

_MESH = dict(core_axis_name="c", subcore_axis_name="s")


def _sc_degrees(e16):
    @functools.partial(
        pl.kernel,
        mesh=plsc.VectorSubcoreMesh(**_MESH),
        out_type=jax.ShapeDtypeStruct((2, NP), jnp.float32),
        scratch_types=[
            pltpu.VMEM((80, 128), jnp.int32),
            pltpu.VMEM((128,), jnp.float32),
            pltpu.VMEM((NPT,), jnp.float32),
            pltpu.VMEM_SHARED((NP,), jnp.float32),
        ],
    )
    def k(e_hbm, out_hbm, idxv, onesv, zv, acc):
        cid = lax.axis_index("c")
        sid = lax.axis_index("s")
        pltpu.sync_copy(e_hbm.at[cid, sid], idxv)
        for i in range(8):
            onesv[pl.ds(i * 16, 16)] = jnp.ones((16,), jnp.float32)

        def zrow(r, c):
            zv[pl.ds(r * 16, 16)] = jnp.zeros((16,), jnp.float32)
            return c

        lax.fori_loop(0, NPT // 16, zrow, 0)
        pltpu.sync_copy(zv, acc.at[pl.ds(sid * NPT, NPT)])
        plsc.subcore_barrier()

        def chunk(j, c):
            pltpu.sync_copy(onesv, acc.at[idxv.at[j]], add=True)
            return c

        lax.fori_loop(0, 80, chunk, 0)
        plsc.subcore_barrier()
        pltpu.sync_copy(acc.at[pl.ds(sid * NPT, NPT)],
                        out_hbm.at[cid, pl.ds(sid * NPT, NPT)])

    return k(e16)


RH = 5120
ACC_R = 5632
ZPT = ACC_R // 16
CPT = RH // 16


def _sc_agg(table, edges, nchunks, per_core_edges, offset_by_core):
    @functools.partial(
        pl.kernel,
        mesh=plsc.VectorSubcoreMesh(**_MESH),
        out_type=jax.ShapeDtypeStruct((2, NP, 128), jnp.float32),
        scratch_types=[
            pltpu.VMEM((nchunks, 128), jnp.int32),
            pltpu.VMEM((nchunks, 128), jnp.int32),
            pltpu.VMEM((nchunks, 128), jnp.int32),
            pltpu.VMEM((128, 128), jnp.float32),
            pltpu.VMEM((128, 128), jnp.float32),
            pltpu.VMEM((32, 128), jnp.float32),
            pltpu.VMEM_SHARED((ACC_R, 128), jnp.float32),
            pltpu.SemaphoreType.DMA,
            pltpu.SemaphoreType.DMA,
        ],
    )
    def k(tab_hbm, e_hbm, out_hbm, srcv, dstAv, dstBv, rows0, rows1,
          zbuf, acc, gsem0, gsem1):
        cid = lax.axis_index("c")
        sid = lax.axis_index("s")
        tid = cid * 16 + sid if per_core_edges else sid
        pltpu.sync_copy(e_hbm.at[0, tid], srcv)
        pltpu.sync_copy(e_hbm.at[1, tid], dstAv)
        pltpu.sync_copy(e_hbm.at[2, tid], dstBv)

        if offset_by_core:
            off = cid * NP

            def addoff(j, c):
                for i in range(8):
                    srcv[j, pl.ds(i * 16, 16)] = srcv[j, pl.ds(i * 16, 16)] + off
                return c

            lax.fori_loop(0, nchunks, addoff, 0)

        def zrow(r, c):
            for i in range(8):
                zbuf[r, pl.ds(i * 16, 16)] = jnp.zeros((16,), jnp.float32)
            return c

        lax.fori_loop(0, 32, zrow, 0)

        for p in range(2):
            dstv = dstAv if p == 0 else dstBv
            for b in range(ZPT // 32):
                pltpu.sync_copy(zbuf,
                                acc.at[pl.ds(sid * ZPT + b * 32, 32)])
            plsc.subcore_barrier()

            pltpu.async_copy(tab_hbm.at[srcv.at[0]], rows0, gsem0)

            def chunk2(jj, c):
                j0 = 2 * jj
                pltpu.async_copy(tab_hbm.at[srcv.at[j0 + 1]], rows1, gsem1)
                pltpu.make_async_copy(tab_hbm.at[srcv.at[j0]], rows0,
                                      gsem0).wait()
                pltpu.sync_copy(rows0, acc.at[dstv.at[j0]], add=True)

                @pl.when(jj < nchunks // 2 - 1)
                def _():
                    pltpu.async_copy(tab_hbm.at[srcv.at[j0 + 2]], rows0, gsem0)

                pltpu.make_async_copy(tab_hbm.at[srcv.at[j0 + 1]], rows1,
                                      gsem1).wait()
                pltpu.sync_copy(rows1, acc.at[dstv.at[j0 + 1]], add=True)
                return c

            lax.fori_loop(0, nchunks // 2, chunk2, 0)
            plsc.subcore_barrier()
            pltpu.sync_copy(acc.at[pl.ds(sid * CPT, CPT)],
                            out_hbm.at[cid, pl.ds(p * RH + sid * CPT, CPT)])
            plsc.subcore_barrier()

    return k(table, edges)


def _tc_prep(degT, Xp):
    bm = 2048

    def body(d_ref, x_ref, xn_ref, n_ref):
        d = d_ref[...]
        nrm = jnp.where(d > 0, lax.rsqrt(jnp.maximum(d, 1.0)), 0.0)
        n_ref[...] = nrm
        ns = nrm[:, 0:1]
        x = x_ref[...]
        xn_ref[0] = x[:, :128] * ns
        xn_ref[1] = x[:, 128:] * ns

    return pl.pallas_call(
        body,
        grid=(NP // bm,),
        in_specs=[
            pl.BlockSpec((bm, 2), lambda i: (i, 0)),
            pl.BlockSpec((bm, 256), lambda i: (i, 0)),
        ],
        out_specs=[
            pl.BlockSpec((2, bm, 128), lambda i: (0, i, 0)),
            pl.BlockSpec((bm, 2), lambda i: (i, 0)),
        ],
        out_shape=[
            jax.ShapeDtypeStruct((2, NP, 128), jnp.float32),
            jax.ShapeDtypeStruct((NP, 2), jnp.float32),
        ],
    )(degT, Xp)


def _tc_mid(agg1, norms, W1a, W1b, b1, W2):
    bm = 2048

    def body(a_ref, n_ref, w1a_ref, w1b_ref, b1_ref, w2_ref, t_ref):
        nrm = n_ref[...]
        ns = nrm[:, 0:1]
        nd = nrm[:, 1:2]
        pre = (jnp.dot(a_ref[0] * nd, w1a_ref[...],
                       preferred_element_type=jnp.float32)
               + jnp.dot(a_ref[1] * nd, w1b_ref[...],
                         preferred_element_type=jnp.float32)
               + b1_ref[...])
        h = jnp.maximum(pre, 0.0)
        t_ref[...] = jnp.dot(h * ns, w2_ref[...],
                             preferred_element_type=jnp.float32)

    return pl.pallas_call(
        body,
        grid=(NP // bm,),
        in_specs=[
            pl.BlockSpec((2, bm, 128), lambda i: (0, i, 0)),
            pl.BlockSpec((bm, 2), lambda i: (i, 0)),
            pl.BlockSpec((128, 256), lambda i: (0, 0)),
            pl.BlockSpec((128, 256), lambda i: (0, 0)),
            pl.BlockSpec((1, 256), lambda i: (0, 0)),
            pl.BlockSpec((256, 128), lambda i: (0, 0)),
        ],
        out_specs=pl.BlockSpec((bm, 128), lambda i: (i, 0)),
        out_shape=jax.ShapeDtypeStruct((NP, 128), jnp.float32),
    )(agg1, norms, W1a, W1b, b1, W2)


def _tc_z(agg2, norms, b2):
    bm = 2048

    def body(a_ref, n_ref, b2_ref, z_ref):
        nd = n_ref[...][:, 1:2]
        z_ref[...] = (a_ref[0] + a_ref[1]) * nd + b2_ref[...]

    return pl.pallas_call(
        body,
        grid=(NP // bm,),
        in_specs=[
            pl.BlockSpec((2, bm, 128), lambda i: (0, i, 0)),
            pl.BlockSpec((bm, 2), lambda i: (i, 0)),
            pl.BlockSpec((1, 128), lambda i: (0, 0)),
        ],
        out_specs=pl.BlockSpec((bm, 128), lambda i: (i, 0)),
        out_shape=jax.ShapeDtypeStruct((NP, 128), jnp.float32),
    )(agg2, norms, b2)


def _tc_decoder(z):
    bm, bn = 1024, 2048

    def body(a_ref, b_ref, o_ref):
        o_ref[...] = lax.dot_general(
            a_ref[...], b_ref[...], (((1,), (1,)), ((), ())),
            preferred_element_type=jnp.float32)

    return pl.pallas_call(
        body,
        grid=(pl.cdiv(N, bm), pl.cdiv(N, bn)),
        in_specs=[
            pl.BlockSpec((bm, 128), lambda i, j: (i, 0)),
            pl.BlockSpec((bn, 128), lambda i, j: (j, 0)),
        ],
        out_specs=pl.BlockSpec((bm, bn), lambda i, j: (i, j)),
        out_shape=jax.ShapeDtypeStruct((N, N), jnp.float32),
    )(z, z)


def kernel(g, features, W1, b1, W2, b2):
    src = g[0].astype(jnp.int32)
    dst = g[1].astype(jnp.int32)
    pad = EP - E
    spread = N + (jnp.arange(pad, dtype=jnp.int32) % (NP - N))
    srcp = jnp.concatenate([src, spread])
    dstp = jnp.concatenate([dst, spread])

    e16 = jnp.stack([srcp, dstp]).reshape(2, 16, 80, 128)
    deg = _sc_degrees(e16)
    degT = deg.T

    Xp = jnp.pad(features, ((0, NP - N), (0, 0)))
    xn, norms = _tc_prep(degT, Xp)

    dum = RH + (jnp.arange(EP, dtype=jnp.int32) & 511)
    dstA = jnp.where(dstp < RH, dstp, dum)
    dstB = jnp.where(dstp >= RH, dstp - RH, dum)
    e3 = jnp.stack([srcp, dstA, dstB])

    table1 = xn.reshape(2 * NP, 128)
    agg1 = _sc_agg(table1, e3.reshape(3, 16, 80, 128), 80,
                   per_core_edges=False, offset_by_core=True)

    t = _tc_mid(agg1, norms, W1[:128], W1[128:], b1.reshape(1, 256), W2)

    agg2 = _sc_agg(t, e3.reshape(3, 32, 40, 128), 40,
                   per_core_edges=True, offset_by_core=False)

    z = _tc_z(agg2, norms, b2.reshape(1, 128))
    return _tc_decoder(z)


# speedup vs baseline: 2.7865x; 1.0268x over previous
import functools

import jax
import jax.numpy as jnp
from jax import lax
from jax.experimental import pallas as pl
from jax.experimental.pallas import tpu as pltpu
from jax.experimental.pallas import tpu_sc as plsc

N = 10000
E = 160000
NP = 10240
EP = 163840
NPT = NP // 16

_MESH = dict(core_axis_name="c", subcore_axis_name="s")


def _sc_degrees(e16):
    @functools.partial(
        pl.kernel,
        mesh=plsc.VectorSubcoreMesh(**_MESH),
        out_type=jax.ShapeDtypeStruct((2, NP), jnp.float32),
        scratch_types=[
            pltpu.VMEM((80, 128), jnp.int32),
            pltpu.VMEM((128,), jnp.float32),
            pltpu.VMEM((NPT,), jnp.float32),
            pltpu.VMEM_SHARED((NP,), jnp.float32),
        ],
    )
    def k(e_hbm, out_hbm, idxv, onesv, zv, acc):
        cid = lax.axis_index("c")
        sid = lax.axis_index("s")
        pltpu.sync_copy(e_hbm.at[cid, sid], idxv)
        for i in range(8):
            onesv[pl.ds(i * 16, 16)] = jnp.ones((16,), jnp.float32)

        def zrow(r, c):
            zv[pl.ds(r * 16, 16)] = jnp.zeros((16,), jnp.float32)
            return c

        lax.fori_loop(0, NPT // 16, zrow, 0)
        pltpu.sync_copy(zv, acc.at[pl.ds(sid * NPT, NPT)])
        plsc.subcore_barrier()

        def chunk(j, c):
            pltpu.sync_copy(onesv, acc.at[idxv.at[j]], add=True)
            return c

        lax.fori_loop(0, 80, chunk, 0)
        plsc.subcore_barrier()
        pltpu.sync_copy(acc.at[pl.ds(sid * NPT, NPT)],
                        out_hbm.at[cid, pl.ds(sid * NPT, NPT)])

    return k(e16)


RH = 5120
ACC_R = 5632
ZPT = ACC_R // 16
CPT = RH // 16


def _sc_agg(table, edges, nchunks, per_core_edges, offset_by_core):
    @functools.partial(
        pl.kernel,
        mesh=plsc.VectorSubcoreMesh(**_MESH),
        out_type=jax.ShapeDtypeStruct((2, NP, 128), jnp.float32),
        scratch_types=[
            pltpu.VMEM((nchunks, 128), jnp.int32),
            pltpu.VMEM((nchunks, 128), jnp.int32),
            pltpu.VMEM((nchunks, 128), jnp.int32),
            pltpu.VMEM((128, 128), jnp.float32),
            pltpu.VMEM((128, 128), jnp.float32),
            pltpu.VMEM((32, 128), jnp.float32),
            pltpu.VMEM_SHARED((ACC_R, 128), jnp.float32),
            pltpu.SemaphoreType.DMA,
            pltpu.SemaphoreType.DMA,
        ],
    )
    def k(tab_hbm, e_hbm, out_hbm, srcv, dstAv, dstBv, rows0, rows1,
          zbuf, acc, gsem0, gsem1):
        cid = lax.axis_index("c")
        sid = lax.axis_index("s")
        tid = cid * 16 + sid if per_core_edges else sid
        pltpu.sync_copy(e_hbm.at[0, tid], srcv)
        pltpu.sync_copy(e_hbm.at[1, tid], dstAv)
        pltpu.sync_copy(e_hbm.at[2, tid], dstBv)

        if offset_by_core:
            off = cid * NP

            def addoff(j, c):
                for i in range(8):
                    srcv[j, pl.ds(i * 16, 16)] = srcv[j, pl.ds(i * 16, 16)] + off
                return c

            lax.fori_loop(0, nchunks, addoff, 0)

        def zrow(r, c):
            for i in range(8):
                zbuf[r, pl.ds(i * 16, 16)] = jnp.zeros((16,), jnp.float32)
            return c

        lax.fori_loop(0, 32, zrow, 0)

        for p in range(2):
            dstv = dstAv if p == 0 else dstBv
            for b in range(ZPT // 32):
                pltpu.sync_copy(zbuf,
                                acc.at[pl.ds(sid * ZPT + b * 32, 32)])
            plsc.subcore_barrier()

            pltpu.async_copy(tab_hbm.at[srcv.at[0]], rows0, gsem0)

            def chunk2(jj, c):
                j0 = 2 * jj
                pltpu.async_copy(tab_hbm.at[srcv.at[j0 + 1]], rows1, gsem1)
                pltpu.make_async_copy(tab_hbm.at[srcv.at[j0]], rows0,
                                      gsem0).wait()
                pltpu.sync_copy(rows0, acc.at[dstv.at[j0]], add=True)

                @pl.when(jj < nchunks // 2 - 1)
                def _():
                    pltpu.async_copy(tab_hbm.at[srcv.at[j0 + 2]], rows0, gsem0)

                pltpu.make_async_copy(tab_hbm.at[srcv.at[j0 + 1]], rows1,
                                      gsem1).wait()
                pltpu.sync_copy(rows1, acc.at[dstv.at[j0 + 1]], add=True)
                return c

            lax.fori_loop(0, nchunks // 2, chunk2, 0)
            plsc.subcore_barrier()
            pltpu.sync_copy(acc.at[pl.ds(sid * CPT, CPT)],
                            out_hbm.at[cid, pl.ds(p * RH + sid * CPT, CPT)])
            plsc.subcore_barrier()

    return k(table, edges)


def _tc_prep(degT, Xp):
    bm = 2048

    def body(d_ref, x_ref, xn_ref, n_ref):
        d = d_ref[...]
        nrm = jnp.where(d > 0, lax.rsqrt(jnp.maximum(d, 1.0)), 0.0)
        n_ref[...] = nrm
        ns = nrm[:, 0:1]
        x = x_ref[...]
        xn_ref[0] = x[:, :128] * ns
        xn_ref[1] = x[:, 128:] * ns

    return pl.pallas_call(
        body,
        grid=(NP // bm,),
        in_specs=[
            pl.BlockSpec((bm, 2), lambda i: (i, 0)),
            pl.BlockSpec((bm, 256), lambda i: (i, 0)),
        ],
        out_specs=[
            pl.BlockSpec((2, bm, 128), lambda i: (0, i, 0)),
            pl.BlockSpec((bm, 2), lambda i: (i, 0)),
        ],
        out_shape=[
            jax.ShapeDtypeStruct((2, NP, 128), jnp.float32),
            jax.ShapeDtypeStruct((NP, 2), jnp.float32),
        ],
    )(degT, Xp)


def _tc_mid(agg1, norms, W1a, W1b, b1, W2):
    bm = 2048

    def body(a_ref, n_ref, w1a_ref, w1b_ref, b1_ref, w2_ref, t_ref):
        nrm = n_ref[...]
        ns = nrm[:, 0:1]
        nd = nrm[:, 1:2]
        pre = (jnp.dot(a_ref[0] * nd, w1a_ref[...],
                       preferred_element_type=jnp.float32)
               + jnp.dot(a_ref[1] * nd, w1b_ref[...],
                         preferred_element_type=jnp.float32)
               + b1_ref[...])
        h = jnp.maximum(pre, 0.0)
        t_ref[...] = jnp.dot(h * ns, w2_ref[...],
                             preferred_element_type=jnp.float32)

    return pl.pallas_call(
        body,
        grid=(NP // bm,),
        in_specs=[
            pl.BlockSpec((2, bm, 128), lambda i: (0, i, 0)),
            pl.BlockSpec((bm, 2), lambda i: (i, 0)),
            pl.BlockSpec((128, 256), lambda i: (0, 0)),
            pl.BlockSpec((128, 256), lambda i: (0, 0)),
            pl.BlockSpec((1, 256), lambda i: (0, 0)),
            pl.BlockSpec((256, 128), lambda i: (0, 0)),
        ],
        out_specs=pl.BlockSpec((bm, 128), lambda i: (i, 0)),
        out_shape=jax.ShapeDtypeStruct((NP, 128), jnp.float32),
    )(agg1, norms, W1a, W1b, b1, W2)


def _tc_z(agg2, norms, b2):
    bm = 2048

    def body(a_ref, n_ref, b2_ref, z_ref):
        nd = n_ref[...][:, 1:2]
        z_ref[...] = (a_ref[0] + a_ref[1]) * nd + b2_ref[...]

    return pl.pallas_call(
        body,
        grid=(NP // bm,),
        in_specs=[
            pl.BlockSpec((2, bm, 128), lambda i: (0, i, 0)),
            pl.BlockSpec((bm, 2), lambda i: (i, 0)),
            pl.BlockSpec((1, 128), lambda i: (0, 0)),
        ],
        out_specs=pl.BlockSpec((bm, 128), lambda i: (i, 0)),
        out_shape=jax.ShapeDtypeStruct((NP, 128), jnp.float32),
    )(agg2, norms, b2)


def _tc_decoder(z):
    bm, bn = 2048, 2048

    def body(a_ref, b_ref, o_ref):
        o_ref[...] = lax.dot_general(
            a_ref[...], b_ref[...], (((1,), (1,)), ((), ())),
            preferred_element_type=jnp.float32)

    return pl.pallas_call(
        body,
        grid=(pl.cdiv(N, bm), pl.cdiv(N, bn)),
        in_specs=[
            pl.BlockSpec((bm, 128), lambda i, j: (i, 0)),
            pl.BlockSpec((bn, 128), lambda i, j: (j, 0)),
        ],
        out_specs=pl.BlockSpec((bm, bn), lambda i, j: (i, j)),
        out_shape=jax.ShapeDtypeStruct((N, N), jnp.float32),
    )(z, z)


def kernel(g, features, W1, b1, W2, b2):
    src = g[0].astype(jnp.int32)
    dst = g[1].astype(jnp.int32)
    pad = EP - E
    spread = N + (jnp.arange(pad, dtype=jnp.int32) % (NP - N))
    srcp = jnp.concatenate([src, spread])
    dstp = jnp.concatenate([dst, spread])

    e16 = jnp.stack([srcp, dstp]).reshape(2, 16, 80, 128)
    deg = _sc_degrees(e16)
    degT = deg.T

    Xp = jnp.pad(features, ((0, NP - N), (0, 0)))
    xn, norms = _tc_prep(degT, Xp)

    dum = RH + (jnp.arange(EP, dtype=jnp.int32) & 511)
    dstA = jnp.where(dstp < RH, dstp, dum)
    dstB = jnp.where(dstp >= RH, dstp - RH, dum)
    e3 = jnp.stack([srcp, dstA, dstB])

    table1 = xn.reshape(2 * NP, 128)
    agg1 = _sc_agg(table1, e3.reshape(3, 16, 80, 128), 80,
                   per_core_edges=False, offset_by_core=True)

    t = _tc_mid(agg1, norms, W1[:128], W1[128:], b1.reshape(1, 256), W2)

    agg2 = _sc_agg(t, e3.reshape(3, 32, 40, 128), 40,
                   per_core_edges=True, offset_by_core=False)

    z = _tc_z(agg2, norms, b2.reshape(1, 128))
    return _tc_decoder(z)
